# baseline (speedup 1.0000x reference)
import functools

import jax
import jax.numpy as jnp
from jax import lax
from jax.experimental import pallas as pl
from jax.experimental.pallas import tpu as pltpu
from jax.experimental.pallas import tpu_sc as plsc

BATCH = 16384
EMBED_DIM = 64
LANES = 16

_info = plsc.get_sparse_core_info()
_NC, _NS = _info.num_cores, _info.num_subcores
NW = _NC * _NS
BPW = BATCH // NW
GROUPS = BPW // LANES


@functools.partial(
    pl.kernel,
    mesh=plsc.VectorSubcoreMesh(core_axis_name="c", subcore_axis_name="s"),
    compiler_params=pltpu.CompilerParams(
        needs_layout_passes=False, use_tc_tiling_on_sc=False
    ),
    out_type=jax.ShapeDtypeStruct((BATCH,), jnp.float32),
    scratch_types=[
        pltpu.VMEM((BPW,), jnp.int32),
        pltpu.VMEM((BPW,), jnp.int32),
        pltpu.VMEM((BPW, EMBED_DIM), jnp.float32),
        pltpu.VMEM((BPW, EMBED_DIM), jnp.float32),
        pltpu.VMEM((BPW,), jnp.float32),
        pltpu.SemaphoreType.DMA,
        pltpu.SemaphoreType.DMA,
    ],
)
def _sc_dot(uidx_hbm, iidx_hbm, utab_hbm, itab_hbm, out_hbm,
            uidx_v, iidx_v, urows_v, irows_v, out_v, sem_u, sem_i):
    wid = lax.axis_index("s") * _NC + lax.axis_index("c")
    base = wid * BPW

    pltpu.sync_copy(uidx_hbm.at[pl.ds(base, BPW)], uidx_v)
    pltpu.sync_copy(iidx_hbm.at[pl.ds(base, BPW)], iidx_v)

    cp_u = pltpu.async_copy(utab_hbm.at[uidx_v], urows_v, sem_u)
    cp_i = pltpu.async_copy(itab_hbm.at[iidx_v], irows_v, sem_i)
    cp_u.wait()
    cp_i.wait()

    lane = lax.iota(jnp.int32, LANES)

    def group_body(g, carry):
        gv = jnp.zeros((LANES,), jnp.float32)
        for j in range(LANES):
            b = g * LANES + j
            acc = jnp.zeros((LANES,), jnp.float32)
            for c in range(EMBED_DIM // LANES):
                uu = urows_v[b, pl.ds(c * LANES, LANES)]
                vv = irows_v[b, pl.ds(c * LANES, LANES)]
                acc = acc + uu * vv
            r = lax.reduce_sum(acc, axes=(0,))
            gv = jnp.where(lane == j, r, gv)
        out_v[pl.ds(g * LANES, LANES)] = gv
        return carry

    lax.fori_loop(0, GROUPS, group_body, 0)

    pltpu.sync_copy(out_v, out_hbm.at[pl.ds(base, BPW)])


def kernel(user_indices, item_indices, user_table, item_table):
    out = _sc_dot(user_indices.astype(jnp.int32),
                  item_indices.astype(jnp.int32),
                  user_table, item_table)
    return out.reshape(BATCH, 1)
